# baseline (speedup 1.0000x reference)
import functools

import jax
import jax.numpy as jnp
from jax import lax
from jax.experimental import pallas as pl
from jax.experimental.pallas import tpu as pltpu
from jax.experimental.pallas import tpu_sc as plsc

NC = 2
NS = 16
L = 16
NW = NC * NS
B = 128
BG = 256
CHG = 4
NUM_LAYERS = 3


def _round_up(v, m):
    return (v + m - 1) // m * m


def _mesh():
    return plsc.VectorSubcoreMesh(
        core_axis_name="c", subcore_axis_name="s",
        num_cores=NC, num_subcores=NS)


@functools.lru_cache(maxsize=None)
def _make_layer_kernel(n, d, nch):
    na = _round_up(n + 1, NS * B)
    rpt = na // NS

    @functools.partial(
        pl.kernel,
        out_type=jax.ShapeDtypeStruct((NC, n, d), jnp.float32),
        mesh=_mesh(),
        scratch_types=[
            pltpu.VMEM((2 * CHG * BG,), jnp.int32),
            pltpu.VMEM((2, 2 * CHG, B), jnp.int32),
            pltpu.VMEM((BG, d), jnp.float32),
            pltpu.VMEM_SHARED((na, d), jnp.float32),
            pltpu.SemaphoreType.DMA,
        ],
    )
    def layerk(src_hbm, dst_hbm, h_hbm, out_hbm,
               sidx, didx, rows, accum, sem_i):
        c = lax.axis_index("c")
        s = lax.axis_index("s")
        w = c * NS + s
        epc = CHG * BG

        pltpu.sync_copy(src_hbm.at[w, 0], sidx.at[pl.ds(0, epc)])
        pltpu.sync_copy(dst_hbm.at[w, 0], didx.at[0])

        nvec = d // L

        @pl.loop(0, BG * nvec)
        def _(i):
            r = i // nvec
            cc = i % nvec
            rows[r, pl.ds(cc * L, L)] = jnp.zeros((L,), jnp.float32)

        base = s * rpt
        for k in range(rpt // BG):
            pltpu.sync_copy(rows, accum.at[pl.ds(base + k * BG, BG)])
        if rpt % BG:
            pltpu.sync_copy(rows.at[pl.ds(0, rpt % BG)],
                            accum.at[pl.ds(base + rpt // BG * BG, rpt % BG)])
        plsc.subcore_barrier()

        @pl.loop(0, nch)
        def _(q):
            p = q % 2
            pn = 1 - p

            @pl.when(q < nch - 1)
            def _():
                pltpu.async_copy(src_hbm.at[w, q + 1],
                                 sidx.at[pl.ds(pn * epc, epc)], sem_i)
                pltpu.async_copy(dst_hbm.at[w, q + 1], didx.at[pn], sem_i)

            for g in range(CHG):
                pltpu.sync_copy(
                    h_hbm.at[sidx.at[pl.ds(p * epc + g * BG, BG)]], rows)
                pltpu.sync_copy(rows.at[pl.ds(0, B)],
                                accum.at[didx.at[p, 2 * g]], add=True)
                pltpu.sync_copy(rows.at[pl.ds(B, B)],
                                accum.at[didx.at[p, 2 * g + 1]], add=True)

            @pl.when(q < nch - 1)
            def _():
                pltpu.make_async_copy(src_hbm.at[w, 0],
                                      sidx.at[pl.ds(0, epc)], sem_i).wait()
                pltpu.make_async_copy(dst_hbm.at[w, 0], didx.at[pn],
                                      sem_i).wait()
        plsc.subcore_barrier()

        nfull = n // rpt
        rem = n - nfull * rpt

        @pl.when(s < nfull)
        def _():
            pltpu.sync_copy(accum.at[pl.ds(base, rpt)],
                            out_hbm.at[c, pl.ds(base, rpt)])
        if rem:
            @pl.when(s == nfull)
            def _():
                pltpu.sync_copy(accum.at[pl.ds(nfull * rpt, rem)],
                                out_hbm.at[c, pl.ds(nfull * rpt, rem)])

    return layerk


@functools.lru_cache(maxsize=None)
def _make_count_kernel(n, kbp):
    nr = -(-(n + 1) // B)

    @functools.partial(
        pl.kernel,
        out_type=jax.ShapeDtypeStruct((NW, nr, B), jnp.float32),
        mesh=_mesh(),
        compiler_params=pltpu.CompilerParams(needs_layout_passes=False),
        scratch_types=[
            pltpu.VMEM((kbp, B), jnp.int32),
            pltpu.VMEM((nr, B), jnp.float32),
        ],
    )
    def countk(dst_hbm, out_hbm, didx, cnt):
        c = lax.axis_index("c")
        s = lax.axis_index("s")
        w = c * NS + s
        pltpu.sync_copy(dst_hbm.at[w], didx)

        @pl.loop(0, nr * (B // L))
        def _(i):
            r = i // (B // L)
            cc = i % (B // L)
            cnt[r, pl.ds(cc * L, L)] = jnp.zeros((L,), jnp.float32)

        @pl.loop(0, kbp * (B // L))
        def _(t):
            j = t // (B // L)
            k = t % (B // L)
            dv = didx[j, pl.ds(k * L, L)]
            plsc.addupdate_scatter(
                cnt, [dv >> 7, dv & 127], jnp.full((L,), 1.0, jnp.float32))

        pltpu.sync_copy(cnt, out_hbm.at[w])

    return countk


def _combine(partials, counts, n, d):
    rb = 1000
    assert n % rb == 0

    def body(p_ref, c_ref, o_ref):
        ssum = p_ref[0] + p_ref[1]
        o_ref[...] = ssum * (1.0 / jnp.maximum(c_ref[...], 1.0))

    return pl.pallas_call(
        body,
        grid=(n // rb,),
        in_specs=[
            pl.BlockSpec((NC, rb, d), lambda i: (0, i, 0)),
            pl.BlockSpec((rb, 1), lambda i: (i, 0)),
        ],
        out_specs=pl.BlockSpec((rb, d), lambda i: (i, 0)),
        out_shape=jax.ShapeDtypeStruct((n, d), jnp.float32),
    )(partials, counts)


def _reduce_counts(counts):
    nw, nf = counts.shape

    def body(c_ref, o_ref):
        o_ref[...] = jnp.sum(c_ref[...], axis=0)[:, None]

    return pl.pallas_call(
        body,
        in_specs=[pl.BlockSpec((nw, nf), lambda: (0, 0))],
        out_specs=pl.BlockSpec((nf, 1), lambda: (0, 0)),
        out_shape=jax.ShapeDtypeStruct((nf, 1), jnp.float32),
    )(counts)


def kernel(x, edge_index):
    n, d = x.shape
    e = edge_index.shape[1]
    src = edge_index[0]
    dst = edge_index[1]

    ew = e // NW
    assert ew * NW == e
    epc = CHG * BG
    nch = -(-ew // epc)
    padn = nch * epc - ew
    src_g = jnp.pad(src.reshape(NW, ew), ((0, 0), (0, padn))
                    ).reshape(NW, nch, CHG * BG)
    dst_g = jnp.pad(dst.reshape(NW, ew), ((0, 0), (0, padn)),
                    constant_values=n).reshape(NW, nch, 2 * CHG, B)

    kbp = nch * epc // B
    dst_p = dst_g.reshape(NW, kbp, B)
    layerk = _make_layer_kernel(n, d, nch)
    counts = _make_count_kernel(n, kbp)(dst_p)
    counts = _reduce_counts(counts.reshape(NW, -1))[:n]
    h = x
    for _ in range(NUM_LAYERS):
        partials = layerk(src_g, dst_g, h)
        h = _combine(partials, counts, n, d)
    return h


# speedup vs baseline: 1.4224x; 1.4224x over previous
import functools

import jax
import jax.numpy as jnp
from jax import lax
from jax.experimental import pallas as pl
from jax.experimental.pallas import tpu as pltpu
from jax.experimental.pallas import tpu_sc as plsc

NC = 2
NS = 16
L = 16
NW = NC * NS
B = 128
NUM_LAYERS = 3


def _round_up(v, m):
    return (v + m - 1) // m * m


def _mesh():
    return plsc.VectorSubcoreMesh(
        core_axis_name="c", subcore_axis_name="s",
        num_cores=NC, num_subcores=NS)


@functools.lru_cache(maxsize=None)
def _make_layer_kernel(n, d, kbp):
    na = _round_up(n + 1, NS * B)
    rpt = na // NS

    @functools.partial(
        pl.kernel,
        out_type=jax.ShapeDtypeStruct((NC, n, d), jnp.float32),
        mesh=_mesh(),
        scratch_types=[
            pltpu.VMEM((kbp, B), jnp.int32),
            pltpu.VMEM((kbp, B), jnp.int32),
            pltpu.VMEM((B, d), jnp.float32),
            pltpu.VMEM_SHARED((na, d), jnp.float32),
        ],
    )
    def layerk(src_hbm, dst_hbm, h_hbm, out_hbm,
               sidx, didx, rows, accum):
        c = lax.axis_index("c")
        s = lax.axis_index("s")
        w = c * NS + s
        pltpu.sync_copy(src_hbm.at[w], sidx)
        pltpu.sync_copy(dst_hbm.at[w], didx)

        nvec = d // L

        @pl.loop(0, B * nvec)
        def _(i):
            r = i // nvec
            cc = i % nvec
            rows[r, pl.ds(cc * L, L)] = jnp.zeros((L,), jnp.float32)

        base = s * rpt
        for k in range(rpt // B):
            pltpu.sync_copy(rows, accum.at[pl.ds(base + k * B, B)])
        plsc.subcore_barrier()

        @pl.loop(0, kbp)
        def _(j):
            pltpu.sync_copy(h_hbm.at[sidx.at[j]], rows)
            pltpu.sync_copy(rows, accum.at[didx.at[j]], add=True)
        plsc.subcore_barrier()

        nfull = n // rpt
        rem = n - nfull * rpt

        @pl.when(s < nfull)
        def _():
            pltpu.sync_copy(accum.at[pl.ds(base, rpt)],
                            out_hbm.at[c, pl.ds(base, rpt)])
        if rem:
            @pl.when(s == nfull)
            def _():
                pltpu.sync_copy(accum.at[pl.ds(nfull * rpt, rem)],
                                out_hbm.at[c, pl.ds(nfull * rpt, rem)])

    return layerk


@functools.lru_cache(maxsize=None)
def _make_count_kernel(n, kbp):
    nr = -(-(n + 1) // B)

    @functools.partial(
        pl.kernel,
        out_type=jax.ShapeDtypeStruct((NW, nr, B), jnp.float32),
        mesh=_mesh(),
        compiler_params=pltpu.CompilerParams(needs_layout_passes=False),
        scratch_types=[
            pltpu.VMEM((kbp, B), jnp.int32),
            pltpu.VMEM((nr, B), jnp.float32),
        ],
    )
    def countk(dst_hbm, out_hbm, didx, cnt):
        c = lax.axis_index("c")
        s = lax.axis_index("s")
        w = c * NS + s
        pltpu.sync_copy(dst_hbm.at[w], didx)

        @pl.loop(0, nr * (B // L))
        def _(i):
            r = i // (B // L)
            cc = i % (B // L)
            cnt[r, pl.ds(cc * L, L)] = jnp.zeros((L,), jnp.float32)

        @pl.loop(0, kbp * (B // L))
        def _(t):
            j = t // (B // L)
            k = t % (B // L)
            dv = didx[j, pl.ds(k * L, L)]
            plsc.addupdate_scatter(
                cnt, [dv >> 7, dv & 127], jnp.full((L,), 1.0, jnp.float32))

        pltpu.sync_copy(cnt, out_hbm.at[w])

    return countk


def _combine(partials, counts, n, d):
    rb = 1000
    assert n % rb == 0

    def body(p_ref, c_ref, o_ref):
        ssum = p_ref[0] + p_ref[1]
        o_ref[...] = ssum * (1.0 / jnp.maximum(c_ref[...], 1.0))

    return pl.pallas_call(
        body,
        grid=(n // rb,),
        in_specs=[
            pl.BlockSpec((NC, rb, d), lambda i: (0, i, 0)),
            pl.BlockSpec((rb, 1), lambda i: (i, 0)),
        ],
        out_specs=pl.BlockSpec((rb, d), lambda i: (i, 0)),
        out_shape=jax.ShapeDtypeStruct((n, d), jnp.float32),
    )(partials, counts)


def _reduce_counts(counts):
    nw, nf = counts.shape

    def body(c_ref, o_ref):
        o_ref[...] = jnp.sum(c_ref[...], axis=0)[:, None]

    return pl.pallas_call(
        body,
        in_specs=[pl.BlockSpec((nw, nf), lambda: (0, 0))],
        out_specs=pl.BlockSpec((nf, 1), lambda: (0, 0)),
        out_shape=jax.ShapeDtypeStruct((nf, 1), jnp.float32),
    )(counts)


def kernel(x, edge_index):
    n, d = x.shape
    e = edge_index.shape[1]
    src = edge_index[0]
    dst = edge_index[1]

    ew = e // NW
    assert ew * NW == e
    kbp = -(-ew // B)
    padn = kbp * B - ew
    src_p = jnp.pad(src.reshape(NW, ew), ((0, 0), (0, padn))
                    ).reshape(NW, kbp, B)
    dst_p = jnp.pad(dst.reshape(NW, ew), ((0, 0), (0, padn)),
                    constant_values=n).reshape(NW, kbp, B)

    layerk = _make_layer_kernel(n, d, kbp)
    counts = _make_count_kernel(n, kbp)(dst_p)
    counts = _reduce_counts(counts.reshape(NW, -1))[:n]
    h = x
    for _ in range(NUM_LAYERS):
        partials = layerk(src_p, dst_p, h)
        h = _combine(partials, counts, n, d)
    return h
